# Initial kernel scaffold; baseline (speedup 1.0000x reference)
import jax
import jax.numpy as jnp
from jax.experimental import pallas as pl


def kernel(x, edge_index, edge_weight, emb_W, emb_b, rel_W0, rel_b0, root_W0, rel_W1, rel_b1, root_W1, rel_W2, rel_b2, root_W2, rel_W3, rel_b3, root_W3, rel_W4, rel_b4, root_W4, lin1_W, lin1_b, lin2_W, lin2_b):
    raise NotImplementedError("write your pallas kernel here")


# speedup vs baseline: 19.2528x; 19.2528x over previous
import functools

import jax
import jax.numpy as jnp
from jax import lax
from jax.experimental import pallas as pl
from jax.experimental.pallas import tpu as pltpu
from jax.experimental.pallas import tpu_sc as plsc

NCORE = 2
NSUB = 16
NW = NCORE * NSUB
LANES = 16
GROUP = 128
KG = 8
CHUNK = GROUP * KG
RB = 2048
PB = RB // 8


def _sc_pass_fn(ntab, nch, zpt):
    mesh = plsc.VectorSubcoreMesh(core_axis_name="c", subcore_axis_name="s")
    nacc = NSUB * zpt

    @functools.partial(
        pl.kernel,
        out_type=jax.ShapeDtypeStruct((NCORE, nacc, LANES), jnp.float32),
        mesh=mesh,
        compiler_params=pltpu.CompilerParams(use_tc_tiling_on_sc=False),
        scratch_types=[
            pltpu.VMEM_SHARED((nacc, LANES), jnp.float32),
            pltpu.VMEM((KG, GROUP), jnp.int32),
            pltpu.VMEM((KG, GROUP), jnp.int32),
            pltpu.VMEM((CHUNK // LANES, LANES), jnp.float32),
            pltpu.VMEM((CHUNK, LANES), jnp.float32),
            pltpu.SemaphoreType.DMA,
            pltpu.SemaphoreType.DMA,
        ],
    )
    def sc_pass(g_hbm, src_hbm, dst_hbm, ew_hbm, z_hbm, out_hbm,
                acc, sidx, didx, ewv, rows, gsem, ssem):
        c = lax.axis_index("c")
        s = lax.axis_index("s")
        tid = c * NSUB + s
        pltpu.sync_copy(z_hbm, acc.at[pl.ds(s * zpt, zpt)])
        plsc.subcore_barrier()

        def chunk_body(ch, carry):
            g0 = ch * KG
            pltpu.sync_copy(src_hbm.at[tid, pl.ds(g0, KG)], sidx)
            pltpu.sync_copy(dst_hbm.at[tid, pl.ds(g0, KG)], didx)
            pltpu.sync_copy(ew_hbm.at[tid, ch], ewv)
            gets = [pltpu.async_copy(g_hbm.at[sidx.at[k]],
                                     rows.at[pl.ds(k * GROUP, GROUP)], gsem)
                    for k in range(KG)]
            for d in gets:
                d.wait()

            @plsc.parallel_loop(0, CHUNK // LANES, 1, unroll=2)
            def _scale(j):
                wv = ewv[j]
                base = j * LANES
                for l in range(LANES):
                    rows[base + l] = rows[base + l] * wv[l]

            puts = [pltpu.async_copy(rows.at[pl.ds(k * GROUP, GROUP)],
                                     acc.at[didx.at[k]], ssem, add=True)
                    for k in range(KG)]
            for d in puts:
                d.wait()
            return carry

        lax.fori_loop(0, nch, chunk_body, None)
        plsc.subcore_barrier()
        pltpu.sync_copy(acc.at[pl.ds(s * zpt, zpt)],
                        out_hbm.at[c, pl.ds(s * zpt, zpt)])

    return sc_pass


def _pad16(w):
    return jnp.pad(w, ((0, 16 - w.shape[0]), (0, 16 - w.shape[1])))


def _k8(w16):
    return jnp.kron(jnp.eye(8, dtype=jnp.float32), w16)


def _b8(b):
    return jnp.tile(jnp.pad(b, (0, 16 - b.shape[0])).reshape(1, 16), (1, 8))


def _full(shape):
    return pl.BlockSpec(shape, lambda i: tuple(0 for _ in shape))


def _stage0_fn(pr):
    grid = (pl.cdiv(pr, PB),)

    def body(x_ref, wga_ref, wgb_ref, wra_ref, wrb_ref,
             ba_ref, bb_ref, bra_ref, brb_ref,
             ga_ref, gb_ref, ra_ref, rb_ref):
        xq = x_ref[...]
        ga_ref[...] = jnp.dot(xq, wga_ref[...],
                              preferred_element_type=jnp.float32) + ba_ref[...]
        gb_ref[...] = jnp.dot(xq, wgb_ref[...],
                              preferred_element_type=jnp.float32) + bb_ref[...]
        ra_ref[...] = jnp.dot(xq, wra_ref[...],
                              preferred_element_type=jnp.float32) + bra_ref[...]
        rb_ref[...] = jnp.dot(xq, wrb_ref[...],
                              preferred_element_type=jnp.float32) + brb_ref[...]

    return pl.pallas_call(
        body,
        grid=grid,
        in_specs=[
            pl.BlockSpec((PB, 160), lambda i: (i, 0)),
            _full((160, 128)), _full((160, 128)),
            _full((160, 128)), _full((160, 128)),
            _full((1, 128)), _full((1, 128)), _full((1, 128)), _full((1, 128)),
        ],
        out_specs=[pl.BlockSpec((PB, 128), lambda i: (i, 0))] * 4,
        out_shape=[jax.ShapeDtypeStruct((pr, 128), jnp.float32)] * 4,
    )


def _stage1_fn(pr):
    grid = (pl.cdiv(pr, PB),)

    def body(pa_ref, pb_ref, ra_ref, rbp_ref, wga_ref, wgb_ref,
             wra_ref, wrb_ref, b_ref, g_ref, r_ref):
        ha = jax.nn.relu(pa_ref[0] + pa_ref[1] + ra_ref[...])
        hb = jax.nn.relu(pb_ref[0] + pb_ref[1] + rbp_ref[...])
        g_ref[...] = (jnp.dot(ha, wga_ref[...], preferred_element_type=jnp.float32)
                      + jnp.dot(hb, wgb_ref[...], preferred_element_type=jnp.float32))
        r_ref[...] = (jnp.dot(ha, wra_ref[...], preferred_element_type=jnp.float32)
                      + jnp.dot(hb, wrb_ref[...], preferred_element_type=jnp.float32)
                      + b_ref[...])

    return pl.pallas_call(
        body,
        grid=grid,
        in_specs=[
            pl.BlockSpec((2, PB, 128), lambda i: (0, i, 0)),
            pl.BlockSpec((2, PB, 128), lambda i: (0, i, 0)),
            pl.BlockSpec((PB, 128), lambda i: (i, 0)),
            pl.BlockSpec((PB, 128), lambda i: (i, 0)),
            _full((128, 128)), _full((128, 128)),
            _full((128, 128)), _full((128, 128)),
            _full((1, 128)),
        ],
        out_specs=[pl.BlockSpec((PB, 128), lambda i: (i, 0))] * 2,
        out_shape=[jax.ShapeDtypeStruct((pr, 128), jnp.float32)] * 2,
    )


def _stage_mid_fn(pr):
    grid = (pl.cdiv(pr, PB),)

    def body(p_ref, rp_ref, wg_ref, wr_ref, b_ref, g_ref, r_ref):
        h = jax.nn.relu(p_ref[0] + p_ref[1] + rp_ref[...])
        g_ref[...] = jnp.dot(h, wg_ref[...], preferred_element_type=jnp.float32)
        r_ref[...] = jnp.dot(h, wr_ref[...],
                             preferred_element_type=jnp.float32) + b_ref[...]

    return pl.pallas_call(
        body,
        grid=grid,
        in_specs=[
            pl.BlockSpec((2, PB, 128), lambda i: (0, i, 0)),
            pl.BlockSpec((PB, 128), lambda i: (i, 0)),
            _full((128, 128)), _full((128, 128)), _full((1, 128)),
        ],
        out_specs=[pl.BlockSpec((PB, 128), lambda i: (i, 0))] * 2,
        out_shape=[jax.ShapeDtypeStruct((pr, 128), jnp.float32)] * 2,
    )


def _stage5_fn(pr):
    grid = (pl.cdiv(pr, PB),)

    def body(p_ref, rp_ref, o_ref):
        o_ref[...] = p_ref[0] + p_ref[1] + rp_ref[...]

    return pl.pallas_call(
        body,
        grid=grid,
        in_specs=[
            pl.BlockSpec((2, PB, 128), lambda i: (0, i, 0)),
            pl.BlockSpec((PB, 128), lambda i: (i, 0)),
        ],
        out_specs=pl.BlockSpec((PB, 128), lambda i: (i, 0)),
        out_shape=jax.ShapeDtypeStruct((pr, 128), jnp.float32),
    )


def _mlp_fn(m):
    def body(a_ref, w1_ref, b1_ref, w2_ref, b2_ref, o_ref):
        h = jax.nn.relu(jnp.dot(a_ref[...], w1_ref[...],
                                preferred_element_type=jnp.float32) + b1_ref[...])
        o_ref[...] = jnp.dot(h, w2_ref[...],
                             preferred_element_type=jnp.float32) + b2_ref[...]

    return pl.pallas_call(
        body,
        out_shape=jax.ShapeDtypeStruct((m, 396), jnp.float32),
    )


def kernel(x, edge_index, edge_weight, emb_W, emb_b, rel_W0, rel_b0, root_W0,
           rel_W1, rel_b1, root_W1, rel_W2, rel_b2, root_W2, rel_W3, rel_b3,
           root_W3, rel_W4, rel_b4, root_W4, lin1_W, lin1_b, lin2_W, lin2_b):
    n = x.shape[0]
    e = edge_weight.shape[0]

    epad = pl.cdiv(e, NW * CHUNK) * NW * CHUNK
    nch = (epad // NW) // CHUNK
    src = jnp.concatenate([edge_index[0],
                           jnp.zeros((epad - e,), jnp.int32)]).reshape(
        NW, nch * KG, GROUP)
    dst = jnp.concatenate([edge_index[1],
                           jnp.zeros((epad - e,), jnp.int32)]).reshape(
        NW, nch * KG, GROUP)
    ew = jnp.concatenate([edge_weight,
                          jnp.zeros((epad - e,), jnp.float32)]).reshape(
        NW, nch, CHUNK // LANES, LANES)

    zpt = pl.cdiv(pl.cdiv(n, NSUB), 8) * 8
    nacc = NSUB * zpt
    pr = nacc // 8
    zeros = jnp.zeros((zpt, LANES), jnp.float32)

    sc_pass = _sc_pass_fn(nacc, nch, zpt)
    unpack = lambda t: t.reshape(nacc, LANES)
    pack_p = lambda p: p.reshape(NCORE, pr, 128)

    eye8 = jnp.eye(8, dtype=jnp.float32)
    embwt = emb_W.T
    embb = emb_b.reshape(1, 20)
    w0t = rel_W0.T
    wga0 = w0t[:, :16]
    wgb0 = jnp.pad(w0t[:, 16:20], ((0, 0), (0, 12)))
    r0t = root_W0.T
    wra0 = r0t[:, :16]
    wrb0 = jnp.pad(r0t[:, 16:20], ((0, 0), (0, 12)))
    b0a = rel_b0[:16].reshape(1, 16)
    b0b = jnp.pad(rel_b0[16:20].reshape(1, 4), ((0, 0), (0, 12)))
    wga0p = jnp.kron(eye8, embwt @ wga0)
    wgb0p = jnp.kron(eye8, embwt @ wgb0)
    wra0p = jnp.kron(eye8, embwt @ wra0)
    wrb0p = jnp.kron(eye8, embwt @ wrb0)
    ba0p = jnp.tile(embb @ wga0, (1, 8))
    bb0p = jnp.tile(embb @ wgb0, (1, 8))
    bra0p = jnp.tile(embb @ wra0 + b0a, (1, 8))
    brb0p = jnp.tile(embb @ wrb0 + b0b, (1, 8))

    w1t = rel_W1.T
    wga1 = _k8(_pad16(w1t[:16]))
    wgb1 = _k8(_pad16(w1t[16:20]))
    r1t = root_W1.T
    wra1 = _k8(_pad16(r1t[:16]))
    wrb1 = _k8(_pad16(r1t[16:20]))
    b1 = _b8(rel_b1)

    wg2, wr2, b2 = _k8(_pad16(rel_W2.T)), _k8(_pad16(root_W2.T)), _b8(rel_b2)
    wg3, wr3, b3 = _k8(_pad16(rel_W3.T)), _k8(_pad16(root_W3.T)), _b8(rel_b3)
    wg4, wr4, b4 = _k8(_pad16(rel_W4.T)), _k8(_pad16(root_W4.T)), _b8(rel_b4)

    xq = jnp.pad(x, ((0, nacc - n), (0, 0))).reshape(pr, 160)
    ga0, gb0, roota0, rootb0 = _stage0_fn(pr)(
        xq, wga0p, wgb0p, wra0p, wrb0p, ba0p, bb0p, bra0p, brb0p)

    pa = pack_p(sc_pass(unpack(ga0), src, dst, ew, zeros))
    pb = pack_p(sc_pass(unpack(gb0), src, dst, ew, zeros))

    g1, root1 = _stage1_fn(pr)(
        pa, pb, roota0, rootb0, wga1, wgb1, wra1, wrb1, b1)

    p1 = pack_p(sc_pass(unpack(g1), src, dst, ew, zeros))
    g2, root2 = _stage_mid_fn(pr)(p1, root1, wg2, wr2, b2)

    p2 = pack_p(sc_pass(unpack(g2), src, dst, ew, zeros))
    g3, root3 = _stage_mid_fn(pr)(p2, root2, wg3, wr3, b3)

    p3 = pack_p(sc_pass(unpack(g3), src, dst, ew, zeros))
    g4, root4 = _stage_mid_fn(pr)(p3, root3, wg4, wr4, b4)

    p4 = pack_p(sc_pass(unpack(g4), src, dst, ew, zeros))
    h5 = _stage5_fn(pr)(p4, root4)

    m = (n * 2) // 396
    flat = h5.reshape(nacc, LANES)[:n, :2].reshape(m, 396)
    out = _mlp_fn(m)(flat, lin1_W.T, lin1_b.reshape(1, 396),
                     lin2_W.T, lin2_b.reshape(1, 396))
    return out.reshape(n, 2)
